# baseline (device time: 14606 ns/iter reference)
import jax
import jax.numpy as jnp
from jax import lax
from jax.experimental import pallas as pl
from jax.experimental.pallas import tpu as pltpu

N_DEV = 4


def kernel(x, router_W, route_idx, expert_W):
    n_tok, d_model = x.shape
    e_per, _, d_out = expert_W.shape

    def body(x_ref, idx_ref, ew_ref, out_ref,
             part_a_ref, part_b_ref, buf_a_ref, buf_b_ref,
             send_sems, recv_sems):
        my = lax.axis_index("i")
        partner_a = my ^ 1
        partner_b = 3 - my

        barrier_sem = pltpu.get_barrier_semaphore()
        for p in [partner_a, partner_b]:
            pl.semaphore_signal(
                barrier_sem, inc=1,
                device_id=(p,), device_id_type=pl.DeviceIdType.MESH,
            )
        pl.semaphore_wait(barrier_sem, 2)

        xv = x_ref[:, :]
        idx = idx_ref[:, :]
        acc = jnp.zeros((n_tok, d_out), jnp.float32)
        for k in range(e_per):
            e = my * e_per + k
            mask = (idx == e).astype(jnp.float32)
            acc = acc + jnp.dot(
                xv * mask, ew_ref[k], preferred_element_type=jnp.float32
            )
        part_a_ref[:, :] = acc

        rdma_a = pltpu.make_async_remote_copy(
            src_ref=part_a_ref,
            dst_ref=buf_a_ref,
            send_sem=send_sems.at[0],
            recv_sem=recv_sems.at[0],
            device_id=(partner_a,),
            device_id_type=pl.DeviceIdType.MESH,
        )
        rdma_a.start()
        rdma_a.wait_recv()
        acc = acc + buf_a_ref[:, :]
        part_b_ref[:, :] = acc

        rdma_b = pltpu.make_async_remote_copy(
            src_ref=part_b_ref,
            dst_ref=buf_b_ref,
            send_sem=send_sems.at[1],
            recv_sem=recv_sems.at[1],
            device_id=(partner_b,),
            device_id_type=pl.DeviceIdType.MESH,
        )
        rdma_b.start()
        rdma_b.wait_recv()
        out_ref[:, :] = acc + buf_b_ref[:, :]

        rdma_a.wait_send()
        rdma_b.wait_send()

    return pl.pallas_call(
        body,
        out_shape=jax.ShapeDtypeStruct((n_tok, d_out), jnp.float32),
        in_specs=[
            pl.BlockSpec(memory_space=pltpu.VMEM),
            pl.BlockSpec(memory_space=pltpu.VMEM),
            pl.BlockSpec(memory_space=pltpu.VMEM),
        ],
        out_specs=pl.BlockSpec(memory_space=pltpu.VMEM),
        scratch_shapes=[
            pltpu.VMEM((n_tok, d_out), jnp.float32),
            pltpu.VMEM((n_tok, d_out), jnp.float32),
            pltpu.VMEM((n_tok, d_out), jnp.float32),
            pltpu.VMEM((n_tok, d_out), jnp.float32),
            pltpu.SemaphoreType.DMA((2,)),
            pltpu.SemaphoreType.DMA((2,)),
        ],
        compiler_params=pltpu.CompilerParams(collective_id=0),
    )(x, route_idx, expert_W)


# device time: 11428 ns/iter; 1.2781x vs baseline; 1.2781x over previous
import jax
import jax.numpy as jnp
from jax import lax
from jax.experimental import pallas as pl
from jax.experimental.pallas import tpu as pltpu

N_DEV = 4
CAP = 128


def kernel(x, router_W, route_idx, expert_W):
    n_tok, d_model = x.shape
    e_per, _, d_out = expert_W.shape

    def body(x_ref, idx_ref, ew_ref, out_ref,
             part_ref, comm_ref, send_sems, recv_sems):
        my = lax.axis_index("i")
        peers = [(my + 1) % N_DEV, (my + 3) % N_DEV, (my + 2) % N_DEV]

        barrier_sem = pltpu.get_barrier_semaphore()
        for p in peers:
            pl.semaphore_signal(
                barrier_sem, inc=1,
                device_id=(p,), device_id_type=pl.DeviceIdType.MESH,
            )
        pl.semaphore_wait(barrier_sem, N_DEV - 1)

        xv = x_ref[:, :]
        idx = idx_ref[:, :]
        acc = jnp.zeros((n_tok, d_out), jnp.float32)
        for k in range(e_per):
            e = my * e_per + k
            mask = (idx == e).astype(jnp.float32)
            acc = acc + jnp.dot(
                xv * mask, ew_ref[k], preferred_element_type=jnp.float32
            )

        d_tok = idx // e_per
        dev_iota = lax.broadcasted_iota(jnp.int32, (n_tok, N_DEV), 1)
        H = (d_tok == dev_iota).astype(jnp.float32)
        same = lax.dot_general(
            H, H, (((1,), (1,)), ((), ())),
            preferred_element_type=jnp.float32,
        )
        r_iota = lax.broadcasted_iota(jnp.int32, (n_tok, n_tok), 0)
        c_iota = lax.broadcasted_iota(jnp.int32, (n_tok, n_tok), 1)
        lower = (r_iota >= c_iota).astype(jnp.float32)
        rank = (jnp.dot(
            lower * same, jnp.ones((n_tok, 1), jnp.float32),
            preferred_element_type=jnp.float32,
        ) - 1.0).astype(jnp.int32)

        cap_cols = lax.broadcasted_iota(jnp.int32, (n_tok, CAP), 1)

        def sel_T(dev):
            m = (d_tok == dev).astype(jnp.float32)
            return (cap_cols == rank).astype(jnp.float32) * m

        part_ref[:, :] = lax.dot_general(
            sel_T(my), acc, (((0,), (0,)), ((), ())),
            preferred_element_type=jnp.float32,
        )

        sends = []
        for j, p in enumerate(peers):
            rdma = pltpu.make_async_remote_copy(
                src_ref=part_ref,
                dst_ref=comm_ref.at[j],
                send_sem=send_sems.at[j],
                recv_sem=recv_sems.at[j],
                device_id=(p,),
                device_id_type=pl.DeviceIdType.MESH,
            )
            rdma.start()
            sends.append(rdma)

        slot_senders = [(my + 3) % N_DEV, (my + 1) % N_DEV, (my + 2) % N_DEV]
        scatters = [sel_T(s) for s in slot_senders]

        for j in range(N_DEV - 1):
            recv = pltpu.make_async_remote_copy(
                src_ref=part_ref,
                dst_ref=comm_ref.at[j],
                send_sem=send_sems.at[j],
                recv_sem=recv_sems.at[j],
                device_id=(peers[j],),
                device_id_type=pl.DeviceIdType.MESH,
            )
            recv.wait_recv()
        for j in range(N_DEV - 1):
            v = comm_ref[j, :, :]
            pin = jnp.minimum(jnp.sum(jnp.abs(v)), 0.0)
            acc = (acc + pin) + jnp.dot(
                scatters[j], v, preferred_element_type=jnp.float32
            )
        out_ref[:, :] = acc

        for rdma in sends:
            rdma.wait_send()

    return pl.pallas_call(
        body,
        out_shape=jax.ShapeDtypeStruct((n_tok, d_out), jnp.float32),
        in_specs=[
            pl.BlockSpec(memory_space=pltpu.VMEM),
            pl.BlockSpec(memory_space=pltpu.VMEM),
            pl.BlockSpec(memory_space=pltpu.VMEM),
        ],
        out_specs=pl.BlockSpec(memory_space=pltpu.VMEM),
        scratch_shapes=[
            pltpu.VMEM((CAP, d_out), jnp.float32),
            pltpu.VMEM((N_DEV - 1, CAP, d_out), jnp.float32),
            pltpu.SemaphoreType.DMA((N_DEV - 1,)),
            pltpu.SemaphoreType.DMA((N_DEV - 1,)),
        ],
        compiler_params=pltpu.CompilerParams(collective_id=0),
    )(x, route_idx, expert_W)


# device time: 9976 ns/iter; 1.4641x vs baseline; 1.1455x over previous
import jax
import jax.numpy as jnp
from jax import lax
from jax.experimental import pallas as pl
from jax.experimental.pallas import tpu as pltpu

N_DEV = 4
CAP = 128


def kernel(x, router_W, route_idx, expert_W):
    n_tok, d_model = x.shape
    e_per, _, d_out = expert_W.shape

    def body(x_ref, idx_ref, ew_ref, out_ref,
             part_ref, comm_ref, send_sems, recv_sems):
        my = lax.axis_index("i")
        peers = [(my + 1) % N_DEV, (my + 3) % N_DEV, (my + 2) % N_DEV]

        barrier_sem = pltpu.get_barrier_semaphore()
        for p in peers:
            pl.semaphore_signal(
                barrier_sem, inc=1,
                device_id=(p,), device_id_type=pl.DeviceIdType.MESH,
            )
        pl.semaphore_wait(barrier_sem, N_DEV - 1)

        idx = idx_ref[:, :]
        d_tok = idx // e_per

        dev_iota = lax.broadcasted_iota(jnp.int32, (n_tok, N_DEV), 1)
        H = (d_tok == dev_iota).astype(jnp.float32)
        r_iota = lax.broadcasted_iota(jnp.int32, (n_tok, n_tok), 0)
        c_iota = lax.broadcasted_iota(jnp.int32, (n_tok, n_tok), 1)
        lower = (r_iota >= c_iota).astype(jnp.float32)
        ranks_all = jnp.dot(lower, H, preferred_element_type=jnp.float32)
        rank = (jnp.sum(H * ranks_all, axis=1, keepdims=True)
                - 1.0).astype(jnp.int32)

        cap_cols = lax.broadcasted_iota(jnp.int32, (n_tok, CAP), 1)

        def sel_T(dev):
            m = (d_tok == dev).astype(jnp.float32)
            return (cap_cols == rank).astype(jnp.float32) * m

        T_my = sel_T(my)

        compact_x = lax.dot_general(
            T_my, x_ref[:, :], (((0,), (0,)), ((), ())),
            preferred_element_type=jnp.float32,
        )
        compact_e = lax.dot_general(
            T_my, idx.astype(jnp.float32), (((0,), (0,)), ((), ())),
            preferred_element_type=jnp.float32,
        )
        pay = jnp.zeros((CAP, d_out), jnp.float32)
        for k in range(e_per):
            e = (my * e_per + k).astype(jnp.float32)
            mask = (compact_e == e).astype(jnp.float32)
            pay = pay + jnp.dot(
                (compact_x * mask).astype(jnp.bfloat16),
                ew_ref[k].astype(jnp.bfloat16),
                preferred_element_type=jnp.float32,
            )
        part_ref[:, :] = pay.astype(jnp.bfloat16)

        sends = []
        for j, p in enumerate(peers):
            rdma = pltpu.make_async_remote_copy(
                src_ref=part_ref,
                dst_ref=comm_ref.at[j],
                send_sem=send_sems.at[j],
                recv_sem=recv_sems.at[j],
                device_id=(p,),
                device_id_type=pl.DeviceIdType.MESH,
            )
            rdma.start()
            sends.append(rdma)

        acc = jnp.dot(T_my, pay, preferred_element_type=jnp.float32)
        slot_senders = [(my + 3) % N_DEV, (my + 1) % N_DEV, (my + 2) % N_DEV]
        scatters = [sel_T(s).astype(jnp.bfloat16) for s in slot_senders]

        for j in range(N_DEV - 1):
            recv = pltpu.make_async_remote_copy(
                src_ref=part_ref,
                dst_ref=comm_ref.at[j],
                send_sem=send_sems.at[j],
                recv_sem=recv_sems.at[j],
                device_id=(peers[j],),
                device_id_type=pl.DeviceIdType.MESH,
            )
            recv.wait_recv()
            v = comm_ref[j, :, :]
            pin = jnp.minimum(jnp.sum(jnp.abs(v.astype(jnp.float32))), 0.0)
            acc = (acc + pin) + jnp.dot(
                scatters[j], v, preferred_element_type=jnp.float32
            )
        out_ref[:, :] = acc

        for rdma in sends:
            rdma.wait_send()

    return pl.pallas_call(
        body,
        out_shape=jax.ShapeDtypeStruct((n_tok, d_out), jnp.float32),
        in_specs=[
            pl.BlockSpec(memory_space=pltpu.VMEM),
            pl.BlockSpec(memory_space=pltpu.VMEM),
            pl.BlockSpec(memory_space=pltpu.VMEM),
        ],
        out_specs=pl.BlockSpec(memory_space=pltpu.VMEM),
        scratch_shapes=[
            pltpu.VMEM((CAP, d_out), jnp.bfloat16),
            pltpu.VMEM((N_DEV - 1, CAP, d_out), jnp.bfloat16),
            pltpu.SemaphoreType.DMA((N_DEV - 1,)),
            pltpu.SemaphoreType.DMA((N_DEV - 1,)),
        ],
        compiler_params=pltpu.CompilerParams(collective_id=0),
    )(x, route_idx, expert_W)
